# baseline (speedup 1.0000x reference)
import functools

import numpy as np
import jax
import jax.numpy as jnp
from jax import lax
from jax.experimental import pallas as pl
from jax.experimental.pallas import tpu as pltpu
from jax.experimental.pallas import tpu_sc as plsc

_SCALE = 0.5
_L = 16
_F = 2
_T = 2 ** 19
_N_MIN = 16
_B_GROWTH = np.exp(np.log(2048 * _SCALE / _N_MIN) / (_L - 1))
_N_POINTS = 262144
_P1 = np.int32(2654435761 - 2 ** 32)
_P2 = np.int32(805459861)
_MASK = np.int32(_T - 1)

_LEVELS = []
for _l in range(_L):
    _s = _N_MIN * (_B_GROWTH ** _l) - 1.0
    _res = int(np.ceil(_s)) + 1
    _LEVELS.append((np.float32(_s), _res, (_res ** 3) <= _T))

_NC, _NS = 2, 16
_NW = _NC * _NS
_PPW = _N_POINTS // _NW
_CH = 512
_NCHUNK = _PPW // _CH
_NI = 8 * _CH
_NR = _NI // 128


def _enc_body(x0h, x1h, x2h, tabh, ench,
              x0v, x1v, x2v, idxv, obuf, wbuf, rowsv, accv, semg):
    wid = lax.axis_index("s") * _NC + lax.axis_index("c")
    iota = lax.iota(jnp.int32, 16)
    pdup = lax.shift_right_logical(iota, 1)
    ppar = lax.bitwise_and(iota, 1)

    def chunk_body(ci, _):
        base = wid * _PPW + ci * _CH
        pltpu.sync_copy(x0h.at[pl.ds(base, _CH)], x0v)
        pltpu.sync_copy(x1h.at[pl.ds(base, _CH)], x1v)
        pltpu.sync_copy(x2h.at[pl.ds(base, _CH)], x2v)

        for l in range(_L):
            s, res, dense = _LEVELS[l]
            lbase = np.int32(l * _T)

            def pass_a(t, _, s=s, res=res, dense=dense, lbase=lbase):
                xo = t * 16
                xv = x0v[pl.ds(xo, 16)]
                yv = x1v[pl.ds(xo, 16)]
                zv = x2v[pl.ds(xo, 16)]
                posx = xv * s + np.float32(0.5)
                posy = yv * s + np.float32(0.5)
                posz = zv * s + np.float32(0.5)
                px = posx.astype(jnp.int32)
                py = posy.astype(jnp.int32)
                pz = posz.astype(jnp.int32)
                fx = posx - px.astype(jnp.float32)
                fy = posy - py.astype(jnp.float32)
                fz = posz - pz.astype(jnp.float32)
                gx = np.float32(1.0) - fx
                gy = np.float32(1.0) - fy
                gz = np.float32(1.0) - fz
                cx1 = px + np.int32(1)
                if dense:
                    ey0 = py * np.int32(res)
                    ey1 = ey0 + np.int32(res)
                    ez0 = pz * np.int32(res * res)
                    ez1 = ez0 + np.int32(res * res)
                else:
                    ey0 = py * _P1
                    ey1 = ey0 + _P1
                    ez0 = pz * _P2
                    ez1 = ez0 + _P2
                for cz in (0, 1):
                    for cy in (0, 1):
                        for cx in (0, 1):
                            a = cx1 if cx else px
                            b = ey1 if cy else ey0
                            c = ez1 if cz else ez0
                            if dense:
                                idx = jnp.minimum(a + b + c, _MASK)
                            else:
                                idx = lax.bitwise_and(
                                    lax.bitwise_xor(lax.bitwise_xor(a, b), c),
                                    _MASK)
                            idx = idx + lbase
                            corner = cz * 4 + cy * 2 + cx
                            q = corner * _CH + xo
                            idxv[q // 128, pl.ds(lax.rem(q, 128), 16)] = \
                                lax.shift_right_logical(idx, 2)
                            obuf[pl.ds(q, 16)] = lax.shift_left(
                                lax.bitwise_and(idx, 3), 1)
                            wbuf[pl.ds(q, 16)] = (fx if cx else gx) \
                                * (fy if cy else gy) * (fz if cz else gz)
                return 0

            lax.fori_loop(0, _CH // 16, pass_a, 0)

            def fire(r, _):
                pltpu.make_async_copy(
                    tabh.at[idxv.at[r]], rowsv.at[r], semg).start()
                return 0

            lax.fori_loop(0, _NR, fire, 0)

            def drain(r, _):
                pltpu.make_async_copy(
                    tabh.at[idxv.at[0]], rowsv.at[0], semg).wait()
                return 0

            lax.fori_loop(0, _NR, drain, 0)

            def pass_b(g, _, l=l):
                fo = g * 8
                acc = jnp.zeros((16,), jnp.float32)
                for corner in range(8):
                    q = corner * _CH + fo + pdup
                    wv = plsc.load_gather(wbuf, [q])
                    ov = plsc.load_gather(obuf, [q]) + ppar
                    rv = plsc.load_gather(
                        rowsv,
                        [lax.shift_right_logical(q, 7),
                         lax.bitwise_and(q, 127), ov])
                    acc = acc + wv * rv
                plsc.store_scatter(
                    accv, [fo + pdup, ppar + np.int32(2 * l)], acc)
                return 0

            lax.fori_loop(0, _CH // 8, pass_b, 0)

        pltpu.sync_copy(accv, ench.at[pl.ds(base, _CH), :])
        return 0

    lax.fori_loop(0, _NCHUNK, chunk_body, 0)


@jax.jit
def _encode(x0, x1, x2, tab):
    mesh = plsc.VectorSubcoreMesh(core_axis_name="c", subcore_axis_name="s")
    f = pl.kernel(
        _enc_body,
        out_type=jax.ShapeDtypeStruct((_N_POINTS, 2 * _L), jnp.float32),
        mesh=mesh,
        scratch_types=[
            pltpu.VMEM((_CH,), jnp.float32),
            pltpu.VMEM((_CH,), jnp.float32),
            pltpu.VMEM((_CH,), jnp.float32),
            pltpu.VMEM((_NR, 128), jnp.int32),
            pltpu.VMEM((_NI,), jnp.int32),
            pltpu.VMEM((_NI,), jnp.float32),
            pltpu.VMEM((_NR, 128, 8), jnp.float32),
            pltpu.VMEM((_CH, 2 * _L), jnp.float32),
            pltpu.SemaphoreType.DMA,
        ],
        compiler_params=pltpu.CompilerParams(
            needs_layout_passes=False, use_tc_tiling_on_sc=False),
    )
    return f(x0, x1, x2, tab)


_BM = 2048


def _mlp_body(enc_ref, w1_ref, w2_ref, h_ref, sig_ref):
    e = enc_ref[...]
    h1 = jnp.maximum(
        lax.dot_general(e, w1_ref[...], (((1,), (0,)), ((), ())),
                        preferred_element_type=jnp.float32,
                        precision=lax.Precision.HIGHEST), 0.0)
    h = lax.dot_general(h1, w2_ref[...], (((1,), (0,)), ((), ())),
                        preferred_element_type=jnp.float32,
                        precision=lax.Precision.HIGHEST)
    h_ref[...] = h
    sig_ref[...] = jnp.exp(jnp.clip(h[:, 0:1], -15.0, 15.0))


@jax.jit
def _mlp(enc, W1, W2):
    return pl.pallas_call(
        _mlp_body,
        grid=(_N_POINTS // _BM,),
        in_specs=[
            pl.BlockSpec((_BM, 2 * _L), lambda i: (i, 0)),
            pl.BlockSpec((2 * _L, 64), lambda i: (0, 0)),
            pl.BlockSpec((64, 16), lambda i: (0, 0)),
        ],
        out_specs=[
            pl.BlockSpec((_BM, 16), lambda i: (i, 0)),
            pl.BlockSpec((_BM, 1), lambda i: (i, 0)),
        ],
        out_shape=[
            jax.ShapeDtypeStruct((_N_POINTS, 16), jnp.float32),
            jax.ShapeDtypeStruct((_N_POINTS, 1), jnp.float32),
        ],
    )(enc, W1, W2)


def kernel(x, table, W1, W2):
    x0 = x[:, 0]
    x1 = x[:, 1]
    x2 = x[:, 2]
    tab = table.reshape(_L * _T * _F // 8, 8)
    enc = _encode(x0, x1, x2, tab)
    h, sig = _mlp(enc, W1, W2)
    return (sig.reshape(-1), h)
